# baseline (speedup 1.0000x reference)
import functools

import jax
import jax.numpy as jnp
from jax import lax
from jax.experimental import pallas as pl
from jax.experimental.pallas import tpu as pltpu
from jax.experimental.pallas import tpu_sc as plsc

_TCA = 25
_GCA = 9
_KS = (24, 23, 22)
_ITERS = 10
_MAXK = 24
_VOCAB = 1000000
_DIM = 32
_NSCAN = _VOCAB - 1
_PACK = 128 // _DIM
_PROWS = _VOCAB // _PACK
_BLK = 2048
_NBLK = (_PROWS + _BLK - 1) // _BLK
_BIG = 3.0e38


@functools.cache
def _sc_gather_fn():
    mesh = plsc.VectorSubcoreMesh(core_axis_name="c", subcore_axis_name="s")

    @functools.partial(
        pl.kernel,
        out_type=jax.ShapeDtypeStruct((32, 128), jnp.float32),
        mesh=mesh,
        scratch_types=[
            pltpu.VMEM((32,), jnp.int32),
            pltpu.VMEM((32, 128), jnp.float32),
            pltpu.SemaphoreType.DMA,
        ],
    )
    def _sc_gather(idx_hbm, table_hbm, out_hbm, idx_v, rows_v, sem):
        wid = lax.axis_index("s") * 2 + lax.axis_index("c")

        @pl.when(wid == 0)
        def _():
            pltpu.sync_copy(idx_hbm, idx_v)
            pltpu.async_copy(table_hbm.at[idx_v], rows_v, sem).wait()
            pltpu.sync_copy(rows_v, out_hbm)

    return _sc_gather


def _kmeans_run(data, pmask, k):
    jj = lax.broadcasted_iota(jnp.int32, (1, _MAXK), 1)
    kmask = jj < k
    cents0 = data[:_MAXK]
    ones_col = jnp.ones((32, 1), jnp.float32)

    def labels_of(cents):
        diff = data[:, None, :] - cents[None, :, :]
        d2 = jnp.sum(diff * diff, axis=-1)
        d2 = jnp.where(kmask, d2, _BIG)
        m = jnp.min(d2, axis=1, keepdims=True)
        return jnp.min(jnp.where(d2 == m, jj, _MAXK), axis=1, keepdims=True)

    def body(_, cents):
        lab = labels_of(cents)
        onehot = ((lab == jj) & pmask).astype(jnp.float32)
        counts = lax.dot_general(onehot, ones_col,
                                 (((0,), (0,)), ((), ())))
        sums = lax.dot_general(onehot, data,
                               (((0,), (0,)), ((), ())))
        newc = sums / jnp.maximum(counts, 1.0)
        return jnp.where(counts > 0, newc, cents)

    cents = lax.fori_loop(0, _ITERS, body, cents0)
    lab = labels_of(cents)
    onehot = ((lab == jj) & pmask).astype(jnp.float32)
    ii = lax.broadcasted_iota(jnp.int32, (32, 1), 0)
    good = jnp.sum(onehot * (ii < _GCA), axis=0, keepdims=True)
    bad = jnp.sum(onehot * ((ii >= _GCA) & pmask), axis=0, keepdims=True)
    sizes = jnp.where((bad == 0.0) & (good > 0.0), good, 0.0)
    return sizes, cents


def _tc_kmeans_body(gath_ref, grp_ref, cent_ref, len_ref):
    gath = gath_ref[...]
    grp = grp_ref[...]
    data = jnp.zeros((32, _DIM), jnp.float32)
    for g in range(_PACK):
        data = data + jnp.where(grp == g,
                                gath[:, g * _DIM:(g + 1) * _DIM], 0.0)
    pmask = lax.broadcasted_iota(jnp.int32, (32, 1), 0) < _TCA
    jj = lax.broadcasted_iota(jnp.int32, (1, _MAXK), 1)

    ms, args, cents_sel = [], [], []
    for k in _KS:
        sizes, cents = _kmeans_run(data, pmask, k)
        m = jnp.max(sizes)
        arg = jnp.min(jnp.where(sizes == m, jj, _MAXK))
        oh = (jj == arg).astype(jnp.float32)
        csel = lax.dot_general(oh, cents, (((1,), (0,)), ((), ())))
        ms.append(m)
        args.append(arg)
        cents_sel.append(csel)

    gm = jnp.maximum(jnp.maximum(ms[0], ms[1]), ms[2])
    s0 = ms[0] == gm
    s1 = (ms[1] == gm) & (~s0)
    s2 = (ms[2] == gm) & (~s0) & (~s1)
    centroid = (jnp.where(s0, 1.0, 0.0) * cents_sel[0]
                + jnp.where(s1, 1.0, 0.0) * cents_sel[1]
                + jnp.where(s2, 1.0, 0.0) * cents_sel[2])

    r = lax.broadcasted_iota(jnp.int32, (_DIM, 128), 0)
    c = lax.broadcasted_iota(jnp.int32, (_DIM, 128), 1)
    tiler = (r == (c % _DIM)).astype(jnp.float32)
    cent_ref[...] = lax.dot_general(centroid, tiler, (((1,), (0,)), ((), ())))
    len_ref[0, 0] = gm.astype(jnp.int32)


_tc_kmeans = pl.pallas_call(
    _tc_kmeans_body,
    out_shape=(
        jax.ShapeDtypeStruct((1, 128), jnp.float32),
        jax.ShapeDtypeStruct((1, 1), jnp.int32),
    ),
    out_specs=(
        pl.BlockSpec(memory_space=pltpu.VMEM),
        pl.BlockSpec(memory_space=pltpu.SMEM),
    ),
)


def _tc_scan_body(cent_ref, emb_ref, idx_ref, dist_ref, minv, mini):
    i = pl.program_id(0)

    @pl.when(i == 0)
    def _():
        minv[0] = jnp.float32(_BIG)
        mini[0] = 0

    x = emb_ref[...]
    z = x - cent_ref[...]
    z2 = z * z
    r = lax.broadcasted_iota(jnp.int32, (128, _PACK), 0)
    c = lax.broadcasted_iota(jnp.int32, (128, _PACK), 1)
    seg = (r // _DIM == c).astype(jnp.float32)
    d2 = lax.dot_general(z2, seg, (((1,), (0,)), ((), ())))

    rr = lax.broadcasted_iota(jnp.int32, (_BLK, _PACK), 0)
    qq = lax.broadcasted_iota(jnp.int32, (_BLK, _PACK), 1)
    rows = (i * _BLK + rr) * _PACK + qq
    d2 = jnp.where(rows < _NSCAN, d2, _BIG)
    bmin = jnp.min(d2)
    barg = jnp.min(jnp.where(d2 == bmin, rows, _VOCAB))

    @pl.when(bmin < minv[0])
    def _():
        minv[0] = bmin
        mini[0] = barg

    @pl.when(i == _NBLK - 1)
    def _():
        idx_ref[0, 0] = mini[0]
        dist_ref[0, 0] = jnp.sqrt(minv[0])


_tc_scan = pl.pallas_call(
    _tc_scan_body,
    grid=(_NBLK,),
    in_specs=[
        pl.BlockSpec((1, 128), lambda i: (0, 0)),
        pl.BlockSpec((_BLK, 128), lambda i: (i, 0)),
    ],
    out_specs=(
        pl.BlockSpec((1, 1), lambda i: (0, 0), memory_space=pltpu.SMEM),
        pl.BlockSpec((1, 1), lambda i: (0, 0), memory_space=pltpu.SMEM),
    ),
    out_shape=(
        jax.ShapeDtypeStruct((1, 1), jnp.int32),
        jax.ShapeDtypeStruct((1, 1), jnp.float32),
    ),
    scratch_shapes=[
        pltpu.SMEM((1,), jnp.float32),
        pltpu.SMEM((1,), jnp.int32),
    ],
)


def kernel(embeddings, good_idx, bad_idx):
    idx = jnp.concatenate([
        good_idx.astype(jnp.int32),
        bad_idx.astype(jnp.int32),
        jnp.zeros((32 - _TCA,), jnp.int32),
    ])
    emb_p = embeddings.reshape(_PROWS, 128)
    gath = _sc_gather_fn()(idx // _PACK, emb_p)
    grp = (idx % _PACK).reshape(32, 1)
    cent_t, clue_len = _tc_kmeans(gath, grp)
    clue_idx, min_dist = _tc_scan(cent_t, emb_p)
    return clue_idx[0, 0], clue_len[0, 0], min_dist[0, 0]


# speedup vs baseline: 1.0865x; 1.0865x over previous
import functools

import jax
import jax.numpy as jnp
from jax import lax
from jax.experimental import pallas as pl
from jax.experimental.pallas import tpu as pltpu
from jax.experimental.pallas import tpu_sc as plsc

_TCA = 25
_GCA = 9
_KS = (24, 23, 22)
_ITERS = 10
_MAXK = 24
_VOCAB = 1000000
_DIM = 32
_NSCAN = _VOCAB - 1
_PACK = 128 // _DIM
_PROWS = _VOCAB // _PACK
_BLK = 8192
_NBLK = (_VOCAB + _BLK - 1) // _BLK
_BIG = 3.0e38


@functools.cache
def _sc_gather_fn():
    mesh = plsc.VectorSubcoreMesh(core_axis_name="c", subcore_axis_name="s")

    @functools.partial(
        pl.kernel,
        out_type=jax.ShapeDtypeStruct((32, 128), jnp.float32),
        mesh=mesh,
        scratch_types=[
            pltpu.VMEM((32,), jnp.int32),
            pltpu.VMEM((32, 128), jnp.float32),
            pltpu.SemaphoreType.DMA,
        ],
    )
    def _sc_gather(idx_hbm, table_hbm, out_hbm, idx_v, rows_v, sem):
        wid = lax.axis_index("s") * 2 + lax.axis_index("c")

        @pl.when(wid == 0)
        def _():
            pltpu.sync_copy(idx_hbm, idx_v)
            pltpu.async_copy(table_hbm.at[idx_v], rows_v, sem).wait()
            pltpu.sync_copy(rows_v, out_hbm)

    return _sc_gather


def _kmeans_run(data, pmask, k):
    jj = lax.broadcasted_iota(jnp.int32, (1, _MAXK), 1)
    kmask = jj < k
    cents0 = data[:_MAXK]
    ones_col = jnp.ones((32, 1), jnp.float32)

    def labels_of(cents):
        diff = data[:, None, :] - cents[None, :, :]
        d2 = jnp.sum(diff * diff, axis=-1)
        d2 = jnp.where(kmask, d2, _BIG)
        m = jnp.min(d2, axis=1, keepdims=True)
        return jnp.min(jnp.where(d2 == m, jj, _MAXK), axis=1, keepdims=True)

    def body(_, cents):
        lab = labels_of(cents)
        onehot = ((lab == jj) & pmask).astype(jnp.float32)
        counts = lax.dot_general(onehot, ones_col,
                                 (((0,), (0,)), ((), ())))
        sums = lax.dot_general(onehot, data,
                               (((0,), (0,)), ((), ())))
        newc = sums / jnp.maximum(counts, 1.0)
        return jnp.where(counts > 0, newc, cents)

    cents = lax.fori_loop(0, _ITERS, body, cents0)
    lab = labels_of(cents)
    onehot = ((lab == jj) & pmask).astype(jnp.float32)
    ii = lax.broadcasted_iota(jnp.int32, (32, 1), 0)
    good = jnp.sum(onehot * (ii < _GCA), axis=0, keepdims=True)
    bad = jnp.sum(onehot * ((ii >= _GCA) & pmask), axis=0, keepdims=True)
    sizes = jnp.where((bad == 0.0) & (good > 0.0), good, 0.0)
    return sizes, cents


def _tc_kmeans_body(idx_ref, emb_ref, cent_ref, len_ref, rows_v, sem):
    copies = [
        pltpu.make_async_copy(emb_ref.at[pl.ds(idx_ref[j], 1)],
                              rows_v.at[pl.ds(j, 1)], sem)
        for j in range(_TCA)
    ]
    for c in copies:
        c.start()
    for c in copies:
        c.wait()
    pmask = lax.broadcasted_iota(jnp.int32, (32, 1), 0) < _TCA
    data = jnp.where(pmask, rows_v[...], 0.0)
    jj = lax.broadcasted_iota(jnp.int32, (1, _MAXK), 1)

    ms, args, cents_sel = [], [], []
    for k in _KS:
        sizes, cents = _kmeans_run(data, pmask, k)
        m = jnp.max(sizes)
        arg = jnp.min(jnp.where(sizes == m, jj, _MAXK))
        oh = (jj == arg).astype(jnp.float32)
        csel = lax.dot_general(oh, cents, (((1,), (0,)), ((), ())))
        ms.append(m)
        args.append(arg)
        cents_sel.append(csel)

    gm = jnp.maximum(jnp.maximum(ms[0], ms[1]), ms[2])
    s0 = ms[0] == gm
    s1 = (ms[1] == gm) & (~s0)
    s2 = (ms[2] == gm) & (~s0) & (~s1)
    centroid = (jnp.where(s0, 1.0, 0.0) * cents_sel[0]
                + jnp.where(s1, 1.0, 0.0) * cents_sel[1]
                + jnp.where(s2, 1.0, 0.0) * cents_sel[2])
    cent_ref[...] = centroid
    len_ref[0, 0] = gm.astype(jnp.int32)


_tc_kmeans = pl.pallas_call(
    _tc_kmeans_body,
    in_specs=[
        pl.BlockSpec(memory_space=pltpu.SMEM),
        pl.BlockSpec(memory_space=pl.ANY),
    ],
    out_shape=(
        jax.ShapeDtypeStruct((1, _DIM), jnp.float32),
        jax.ShapeDtypeStruct((1, 1), jnp.int32),
    ),
    out_specs=(
        pl.BlockSpec(memory_space=pltpu.VMEM),
        pl.BlockSpec(memory_space=pltpu.SMEM),
    ),
    scratch_shapes=[
        pltpu.VMEM((32, _DIM), jnp.float32),
        pltpu.SemaphoreType.DMA,
    ],
)


def _tc_scan_body(cent_ref, emb_ref, idx_ref, dist_ref, minv, mini):
    i = pl.program_id(0)

    @pl.when(i == 0)
    def _():
        minv[0] = jnp.float32(_BIG)
        mini[0] = 0

    x = emb_ref[...]
    z = x - cent_ref[...]
    z2 = z * z
    ones = jnp.ones((_DIM, 1), jnp.float32)
    d2 = lax.dot_general(z2, ones, (((1,), (0,)), ((), ())))

    rows = i * _BLK + lax.broadcasted_iota(jnp.int32, (_BLK, 1), 0)
    d2 = jnp.where(rows < _NSCAN, d2, _BIG)
    bmin = jnp.min(d2)
    barg = jnp.min(jnp.where(d2 == bmin, rows, _VOCAB))

    @pl.when(bmin < minv[0])
    def _():
        minv[0] = bmin
        mini[0] = barg

    @pl.when(i == _NBLK - 1)
    def _():
        idx_ref[0, 0] = mini[0]
        dist_ref[0, 0] = jnp.sqrt(minv[0])


_tc_scan = pl.pallas_call(
    _tc_scan_body,
    grid=(_NBLK,),
    in_specs=[
        pl.BlockSpec((1, _DIM), lambda i: (0, 0)),
        pl.BlockSpec((_BLK, _DIM), lambda i: (i, 0)),
    ],
    out_specs=(
        pl.BlockSpec((1, 1), lambda i: (0, 0), memory_space=pltpu.SMEM),
        pl.BlockSpec((1, 1), lambda i: (0, 0), memory_space=pltpu.SMEM),
    ),
    out_shape=(
        jax.ShapeDtypeStruct((1, 1), jnp.int32),
        jax.ShapeDtypeStruct((1, 1), jnp.float32),
    ),
    scratch_shapes=[
        pltpu.SMEM((1,), jnp.float32),
        pltpu.SMEM((1,), jnp.int32),
    ],
)


def kernel(embeddings, good_idx, bad_idx):
    idx = jnp.concatenate([
        good_idx.astype(jnp.int32),
        bad_idx.astype(jnp.int32),
        jnp.zeros((32 - _TCA,), jnp.int32),
    ])
    cent, clue_len = _tc_kmeans(idx, embeddings)
    clue_idx, min_dist = _tc_scan(cent, embeddings)
    return clue_idx[0, 0], clue_len[0, 0], min_dist[0, 0]
